# baseline (speedup 1.0000x reference)
import functools

import jax
import jax.numpy as jnp
from jax.experimental import pallas as pl
from jax.experimental.pallas import tpu as pltpu
from jax.experimental.pallas import tpu_sc as plsc


_SC_W = 256


def _sc_gather(table, row_idx, col_idx):
    e = row_idx.shape[1]
    mesh = plsc.VectorSubcoreMesh(core_axis_name="c", subcore_axis_name="s")
    out_type = [
        jax.ShapeDtypeStruct((e, 128), jnp.float32),
        jax.ShapeDtypeStruct((e, 128), jnp.float32),
    ]

    @pl.kernel(out_type=out_type, mesh=mesh)
    def gather_kernel(t_hbm, ri_hbm, ci_hbm, gr_hbm, gc_hbm):
        def body(i_vmem, o_vmem):
            pltpu.sync_copy(t_hbm.at[i_vmem.at[0]], o_vmem)

        for idx_hbm, o_hbm in ((ri_hbm, gr_hbm), (ci_hbm, gc_hbm)):
            pltpu.emit_pipeline(
                body,
                grid=(e // _SC_W,),
                in_specs=[pl.BlockSpec((1, _SC_W), lambda i: (0, i))],
                out_specs=[pl.BlockSpec((_SC_W, 128), lambda i: (i, 0))],
                core_axis_name=("c", "s"),
                dimension_semantics=(pltpu.PARALLEL,),
            )(idx_hbm, o_hbm)

    return gather_kernel(table, row_idx, col_idx)


_SC_SCH = 256
_SC_HALF = 50048
_SC_SLC = _SC_HALF // 16


def _sc_scatter(payload, idx0, idx1, zeros, width, col_block):
    e = payload.shape[0]
    mesh = plsc.VectorSubcoreMesh(core_axis_name="c", subcore_axis_name="s")
    out_type = jax.ShapeDtypeStruct((2 * _SC_HALF, 128), jnp.float32)

    @pl.kernel(out_type=out_type, mesh=mesh,
               compiler_params=pltpu.CompilerParams(use_tc_tiling_on_sc=False),
               scratch_types=[
                   pltpu.VMEM_SHARED((_SC_HALF, width), jnp.float32),
                   pltpu.VMEM((8, width), jnp.float32),
                   pltpu.VMEM((8, 128), jnp.float32)])
    def scatter_kernel(pay_hbm, i0_hbm, i1_hbm, z_hbm, out_hbm,
                       acc, bw, b128):
        c = jax.lax.axis_index("c")
        s = jax.lax.axis_index("s")
        pltpu.sync_copy(z_hbm, bw)

        @pl.loop(0, _SC_SLC // 8)
        def _(k):
            pltpu.sync_copy(bw, acc.at[pl.ds(s * _SC_SLC + k * 8, 8)])

        plsc.subcore_barrier()

        def body(pay_vmem, i0_vmem, i1_vmem):
            @pl.when(c == 0)
            def _():
                pltpu.sync_copy(pay_vmem, acc.at[i0_vmem.at[0]], add=True)

            @pl.when(c == 1)
            def _():
                pltpu.sync_copy(pay_vmem, acc.at[i1_vmem.at[0]], add=True)

        pltpu.emit_pipeline(
            body,
            grid=(e // _SC_SCH,),
            in_specs=[pl.BlockSpec((_SC_SCH, width),
                                   lambda i: (i, col_block)),
                      pl.BlockSpec((1, _SC_SCH), lambda i: (0, i)),
                      pl.BlockSpec((1, _SC_SCH), lambda i: (0, i))],
            out_specs=[],
            core_axis_name="s",
            dimension_semantics=(pltpu.PARALLEL,),
        )(pay_hbm, i0_hbm, i1_hbm)
        plsc.subcore_barrier()

        @pl.loop(0, _SC_SLC // 8)
        def _(k):
            pltpu.sync_copy(acc.at[pl.ds(s * _SC_SLC + k * 8, 8)], bw)
            for r in range(8):
                for j in range(width // 16):
                    b128[r, pl.ds(16 * j, 16)] = bw[r, pl.ds(16 * j, 16)]
            pltpu.sync_copy(
                b128,
                out_hbm.at[pl.ds(c * _SC_HALF + s * _SC_SLC + k * 8, 8)])

    return scatter_kernel(payload, idx0, idx1, zeros)


def _largest_block(total, cap, mult):
    best = mult
    b = mult
    while b <= cap:
        if total % b == 0:
            best = b
        b += mult
    return best


def _edge_mlp_body(gr_ref, gc_ref,
                   We1_ref, be1_ref, We2_ref, be2_ref, Wa_ref, ba_ref,
                   Wc1_ref, bc1_ref, Wc2_ref, out_ref):
    gr = gr_ref[...]
    gc = gc_ref[...]
    rel = gr[:, 16:19] - gc[:, 16:19]
    dist = jnp.sum(rel * rel, axis=-1, keepdims=True)
    ef = jnp.concatenate([gr[:, 0:16], gc[:, 0:16], dist], axis=-1)
    m = jax.nn.silu(jnp.dot(ef, We1_ref[...],
                            preferred_element_type=jnp.float32) + be1_ref[...])
    m = jax.nn.silu(jnp.dot(m, We2_ref[...],
                            preferred_element_type=jnp.float32) + be2_ref[...])
    att = jax.nn.sigmoid(jnp.dot(m, Wa_ref[...],
                                 preferred_element_type=jnp.float32) + ba_ref[...])
    m = m * att
    t = jax.nn.silu(jnp.dot(m, Wc1_ref[...],
                            preferred_element_type=jnp.float32) + bc1_ref[...])
    w = jnp.dot(t, Wc2_ref[...], preferred_element_type=jnp.float32)
    trans = rel * w
    ones = jnp.ones_like(w)
    pad = jnp.zeros((trans.shape[0], 92), jnp.float32)
    out_ref[...] = jnp.concatenate([m, trans, ones, pad], axis=-1)


def _edge_mlp(gr, gc, We1, be1, We2, be2, Wa, ba, Wc1, bc1, Wc2):
    e = gr.shape[0]
    be = _largest_block(e, 2000, 8)
    grid = e // be
    return pl.pallas_call(
        _edge_mlp_body,
        grid=(grid,),
        in_specs=[
            pl.BlockSpec((be, 128), lambda i: (i, 0)),
            pl.BlockSpec((be, 128), lambda i: (i, 0)),
            pl.BlockSpec((33, 32), lambda i: (0, 0)),
            pl.BlockSpec((1, 32), lambda i: (0, 0)),
            pl.BlockSpec((32, 32), lambda i: (0, 0)),
            pl.BlockSpec((1, 32), lambda i: (0, 0)),
            pl.BlockSpec((32, 1), lambda i: (0, 0)),
            pl.BlockSpec((1, 1), lambda i: (0, 0)),
            pl.BlockSpec((32, 32), lambda i: (0, 0)),
            pl.BlockSpec((1, 32), lambda i: (0, 0)),
            pl.BlockSpec((32, 1), lambda i: (0, 0)),
        ],
        out_specs=pl.BlockSpec((be, 128), lambda i: (i, 0)),
        out_shape=jax.ShapeDtypeStruct((e, 128), jnp.float32),
    )(gr, gc, We1, be1.reshape(1, -1), We2, be2.reshape(1, -1),
      Wa, ba.reshape(1, -1), Wc1, bc1.reshape(1, -1), Wc2)


def _node_body(aggm_ref, aggtc_ref, coords_ref, hidden_ref,
               Wn1h_ref, Wn1m_ref, bn1_ref, Wn2_ref, bn2_ref,
               coords_out_ref, hout_ref, stats_ref):
    i = pl.program_id(0)
    msg = aggm_ref[...]
    tc = aggtc_ref[...]
    num = tc[:, 0:3]
    cnt = tc[:, 3:4]
    coords_out_ref[...] = coords_ref[...] + num / jnp.maximum(cnt, 1.0)
    h = hidden_ref[...]
    nh = jax.nn.silu(
        jnp.dot(h, Wn1h_ref[...], preferred_element_type=jnp.float32)
        + jnp.dot(msg, Wn1m_ref[...], preferred_element_type=jnp.float32)
        + bn1_ref[...])
    h_out = h + (jnp.dot(nh, Wn2_ref[...],
                         preferred_element_type=jnp.float32) + bn2_ref[...])
    hout_ref[...] = h_out
    colsum = jnp.sum(h_out, axis=0, keepdims=True)
    sumsq = jnp.sum(h_out * h_out).reshape(1, 1)
    s = jnp.concatenate([colsum, sumsq], axis=1)
    s = jnp.pad(s, ((0, 0), (0, 128 - s.shape[1])))
    blk = jnp.broadcast_to(s, (8, 128))

    @pl.when(i == 0)
    def _():
        stats_ref[...] = jnp.zeros_like(stats_ref)

    stats_ref[...] += blk


def _node_update(aggm, aggtc, coords, hidden, Wn1, bn1, Wn2, bn2):
    n = coords.shape[0]
    bn = _largest_block(n, 2000, 8)
    grid = n // bn
    return pl.pallas_call(
        _node_body,
        grid=(grid,),
        in_specs=[
            pl.BlockSpec((bn, 32), lambda i: (i, 0)),
            pl.BlockSpec((bn, 16), lambda i: (i, 0)),
            pl.BlockSpec((bn, 3), lambda i: (i, 0)),
            pl.BlockSpec((bn, 16), lambda i: (i, 0)),
            pl.BlockSpec((16, 32), lambda i: (0, 0)),
            pl.BlockSpec((32, 32), lambda i: (0, 0)),
            pl.BlockSpec((1, 32), lambda i: (0, 0)),
            pl.BlockSpec((32, 16), lambda i: (0, 0)),
            pl.BlockSpec((1, 16), lambda i: (0, 0)),
        ],
        out_specs=[
            pl.BlockSpec((bn, 3), lambda i: (i, 0)),
            pl.BlockSpec((bn, 16), lambda i: (i, 0)),
            pl.BlockSpec((8, 128), lambda i: (0, 0)),
        ],
        out_shape=[
            jax.ShapeDtypeStruct((n, 3), jnp.float32),
            jax.ShapeDtypeStruct((n, 16), jnp.float32),
            jax.ShapeDtypeStruct((8, 128), jnp.float32),
        ],
    )(aggm, aggtc, coords, hidden, Wn1[:16], Wn1[16:], bn1.reshape(1, -1),
      Wn2, bn2.reshape(1, -1))


def _norm_body(hout_ref, stats_ref, out_ref, *, n):
    s = stats_ref[...]
    colsum = s[0:1, 0:16]
    sumsq = s[0:1, 16:17]
    mu = colsum / n
    var = sumsq / n - jnp.sum(mu * mu).reshape(1, 1)
    inv = jax.lax.rsqrt(1e-6 + var)
    out_ref[...] = (hout_ref[...] - mu) * inv


def _normalize(hout, stats):
    n = hout.shape[0]
    bn = _largest_block(n, 4000, 8)
    grid = n // bn
    return pl.pallas_call(
        functools.partial(_norm_body, n=float(n)),
        grid=(grid,),
        in_specs=[
            pl.BlockSpec((bn, 16), lambda i: (i, 0)),
            pl.BlockSpec((8, 128), lambda i: (0, 0)),
        ],
        out_specs=pl.BlockSpec((bn, 16), lambda i: (i, 0)),
        out_shape=jax.ShapeDtypeStruct((n, 16), jnp.float32),
    )(hout, stats)


def kernel(batch_coords, batch_hidden, edges, We1, be1, We2, be2, Wa, ba,
           Wc1, bc1, Wc2, Wn1, bn1, Wn2, bn2):
    n = batch_coords.shape[0]
    row = edges[0]
    table = jnp.pad(jnp.concatenate([batch_hidden, batch_coords], axis=1),
                    ((0, 0), (0, 109)))
    gr, gc = _sc_gather(table, edges[0:1], edges[1:2])
    payload = _edge_mlp(gr, gc, We1, be1, We2, be2, Wa, ba,
                        Wc1, bc1, Wc2)
    half = n // 2
    trash = jnp.int32(half)
    idx0 = jnp.where(row < half, row, trash).reshape(1, -1)
    idx1 = jnp.where(row >= half, row - half, trash).reshape(1, -1)
    zeros32 = jnp.zeros((8, 32), jnp.float32)
    zeros16 = jnp.zeros((8, 16), jnp.float32)
    aggpm = _sc_scatter(payload, idx0, idx1, zeros32, 32, 0)
    aggptc = _sc_scatter(payload, idx0, idx1, zeros16, 16, 2)
    aggm = jnp.concatenate([aggpm[:half, :32],
                            aggpm[_SC_HALF:_SC_HALF + half, :32]], axis=0)
    aggtc = jnp.concatenate([aggptc[:half, :16],
                             aggptc[_SC_HALF:_SC_HALF + half, :16]], axis=0)
    coords_out, h_out, stats = _node_update(
        aggm, aggtc, batch_coords, batch_hidden, Wn1, bn1, Wn2, bn2)
    h_norm = _normalize(h_out, stats)
    return coords_out, h_norm


# speedup vs baseline: 1.9008x; 1.9008x over previous
import functools

import jax
import jax.numpy as jnp
from jax.experimental import pallas as pl
from jax.experimental.pallas import tpu as pltpu
from jax.experimental.pallas import tpu_sc as plsc


_SC_W = 128


def _sc_gather(table, row_idx, col_idx):
    e = row_idx.shape[1]
    mesh = plsc.VectorSubcoreMesh(core_axis_name="c", subcore_axis_name="s")
    out_type = [
        jax.ShapeDtypeStruct((e, 128), jnp.float32),
        jax.ShapeDtypeStruct((e, 128), jnp.float32),
    ]

    @pl.kernel(out_type=out_type, mesh=mesh)
    def gather_kernel(t_hbm, ri_hbm, ci_hbm, gr_hbm, gc_hbm):
        def body(i_vmem, o_vmem):
            pltpu.sync_copy(t_hbm.at[i_vmem.at[0]], o_vmem)

        for idx_hbm, o_hbm in ((ri_hbm, gr_hbm), (ci_hbm, gc_hbm)):
            pltpu.emit_pipeline(
                body,
                grid=(e // _SC_W,),
                in_specs=[pl.BlockSpec((1, _SC_W), lambda i: (0, i))],
                out_specs=[pl.BlockSpec((_SC_W, 128), lambda i: (i, 0))],
                core_axis_name=("c", "s"),
                dimension_semantics=(pltpu.PARALLEL,),
            )(idx_hbm, o_hbm)

    return gather_kernel(table, row_idx, col_idx)


_SC_SCH = 128
_SC_HALF = 50048
_SC_SLC = _SC_HALF // 16


def _sc_scatter(payload, idx0, idx1, zeros, width, col_block):
    e = payload.shape[0]
    mesh = plsc.VectorSubcoreMesh(core_axis_name="c", subcore_axis_name="s")
    out_type = jax.ShapeDtypeStruct((2 * _SC_HALF, 128), jnp.float32)

    @pl.kernel(out_type=out_type, mesh=mesh,
               compiler_params=pltpu.CompilerParams(use_tc_tiling_on_sc=False),
               scratch_types=[
                   pltpu.VMEM_SHARED((_SC_HALF, width), jnp.float32),
                   pltpu.VMEM((8, width), jnp.float32),
                   pltpu.VMEM((8, 128), jnp.float32)])
    def scatter_kernel(pay_hbm, i0_hbm, i1_hbm, z_hbm, out_hbm,
                       acc, bw, b128):
        c = jax.lax.axis_index("c")
        s = jax.lax.axis_index("s")
        pltpu.sync_copy(z_hbm, bw)

        @pl.loop(0, _SC_SLC // 8)
        def _(k):
            pltpu.sync_copy(bw, acc.at[pl.ds(s * _SC_SLC + k * 8, 8)])

        plsc.subcore_barrier()

        def body(pay_vmem, i0_vmem, i1_vmem):
            @pl.when(c == 0)
            def _():
                pltpu.sync_copy(pay_vmem, acc.at[i0_vmem.at[0]], add=True)

            @pl.when(c == 1)
            def _():
                pltpu.sync_copy(pay_vmem, acc.at[i1_vmem.at[0]], add=True)

        pltpu.emit_pipeline(
            body,
            grid=(e // _SC_SCH,),
            in_specs=[pl.BlockSpec((_SC_SCH, width),
                                   lambda i: (i, col_block)),
                      pl.BlockSpec((1, _SC_SCH), lambda i: (0, i)),
                      pl.BlockSpec((1, _SC_SCH), lambda i: (0, i))],
            out_specs=[],
            core_axis_name="s",
            dimension_semantics=(pltpu.PARALLEL,),
        )(pay_hbm, i0_hbm, i1_hbm)
        plsc.subcore_barrier()

        @pl.loop(0, _SC_SLC // 8)
        def _(k):
            pltpu.sync_copy(acc.at[pl.ds(s * _SC_SLC + k * 8, 8)], bw)
            for r in range(8):
                for j in range(width // 16):
                    b128[r, pl.ds(16 * j, 16)] = bw[r, pl.ds(16 * j, 16)]
            pltpu.sync_copy(
                b128,
                out_hbm.at[pl.ds(c * _SC_HALF + s * _SC_SLC + k * 8, 8)])

    return scatter_kernel(payload, idx0, idx1, zeros)


def _largest_block(total, cap, mult):
    best = mult
    b = mult
    while b <= cap:
        if total % b == 0:
            best = b
        b += mult
    return best


def _edge_mlp_body(gr_ref, gc_ref,
                   We1_ref, be1_ref, We2_ref, be2_ref, Wa_ref, ba_ref,
                   Wc1_ref, bc1_ref, Wc2_ref, out_ref):
    gr = gr_ref[...]
    gc = gc_ref[...]
    rel = gr[:, 16:19] - gc[:, 16:19]
    dist = jnp.sum(rel * rel, axis=-1, keepdims=True)
    ef = jnp.concatenate([gr[:, 0:16], gc[:, 0:16], dist], axis=-1)
    m = jax.nn.silu(jnp.dot(ef, We1_ref[...],
                            preferred_element_type=jnp.float32) + be1_ref[...])
    m = jax.nn.silu(jnp.dot(m, We2_ref[...],
                            preferred_element_type=jnp.float32) + be2_ref[...])
    att = jax.nn.sigmoid(jnp.dot(m, Wa_ref[...],
                                 preferred_element_type=jnp.float32) + ba_ref[...])
    m = m * att
    t = jax.nn.silu(jnp.dot(m, Wc1_ref[...],
                            preferred_element_type=jnp.float32) + bc1_ref[...])
    w = jnp.dot(t, Wc2_ref[...], preferred_element_type=jnp.float32)
    trans = rel * w
    ones = jnp.ones_like(w)
    pad = jnp.zeros((trans.shape[0], 92), jnp.float32)
    out_ref[...] = jnp.concatenate([m, trans, ones, pad], axis=-1)


def _edge_mlp(gr, gc, We1, be1, We2, be2, Wa, ba, Wc1, bc1, Wc2):
    e = gr.shape[0]
    be = _largest_block(e, 2000, 8)
    grid = e // be
    return pl.pallas_call(
        _edge_mlp_body,
        grid=(grid,),
        in_specs=[
            pl.BlockSpec((be, 128), lambda i: (i, 0)),
            pl.BlockSpec((be, 128), lambda i: (i, 0)),
            pl.BlockSpec((33, 32), lambda i: (0, 0)),
            pl.BlockSpec((1, 32), lambda i: (0, 0)),
            pl.BlockSpec((32, 32), lambda i: (0, 0)),
            pl.BlockSpec((1, 32), lambda i: (0, 0)),
            pl.BlockSpec((32, 1), lambda i: (0, 0)),
            pl.BlockSpec((1, 1), lambda i: (0, 0)),
            pl.BlockSpec((32, 32), lambda i: (0, 0)),
            pl.BlockSpec((1, 32), lambda i: (0, 0)),
            pl.BlockSpec((32, 1), lambda i: (0, 0)),
        ],
        out_specs=pl.BlockSpec((be, 128), lambda i: (i, 0)),
        out_shape=jax.ShapeDtypeStruct((e, 128), jnp.float32),
    )(gr, gc, We1, be1.reshape(1, -1), We2, be2.reshape(1, -1),
      Wa, ba.reshape(1, -1), Wc1, bc1.reshape(1, -1), Wc2)


def _node_body(aggm_ref, aggtc_ref, coords_ref, hidden_ref,
               Wn1h_ref, Wn1m_ref, bn1_ref, Wn2_ref, bn2_ref,
               coords_out_ref, hout_ref, stats_ref):
    i = pl.program_id(0)
    msg = aggm_ref[...]
    tc = aggtc_ref[...]
    num = tc[:, 0:3]
    cnt = tc[:, 3:4]
    coords_out_ref[...] = coords_ref[...] + num / jnp.maximum(cnt, 1.0)
    h = hidden_ref[...]
    nh = jax.nn.silu(
        jnp.dot(h, Wn1h_ref[...], preferred_element_type=jnp.float32)
        + jnp.dot(msg, Wn1m_ref[...], preferred_element_type=jnp.float32)
        + bn1_ref[...])
    h_out = h + (jnp.dot(nh, Wn2_ref[...],
                         preferred_element_type=jnp.float32) + bn2_ref[...])
    hout_ref[...] = h_out
    colsum = jnp.sum(h_out, axis=0, keepdims=True)
    sumsq = jnp.sum(h_out * h_out).reshape(1, 1)
    s = jnp.concatenate([colsum, sumsq], axis=1)
    s = jnp.pad(s, ((0, 0), (0, 128 - s.shape[1])))
    blk = jnp.broadcast_to(s, (8, 128))

    @pl.when(i == 0)
    def _():
        stats_ref[...] = jnp.zeros_like(stats_ref)

    stats_ref[...] += blk


def _node_update(aggm, aggtc, coords, hidden, Wn1, bn1, Wn2, bn2):
    n = coords.shape[0]
    bn = _largest_block(n, 2000, 8)
    grid = n // bn
    return pl.pallas_call(
        _node_body,
        grid=(grid,),
        in_specs=[
            pl.BlockSpec((bn, 32), lambda i: (i, 0)),
            pl.BlockSpec((bn, 16), lambda i: (i, 0)),
            pl.BlockSpec((bn, 3), lambda i: (i, 0)),
            pl.BlockSpec((bn, 16), lambda i: (i, 0)),
            pl.BlockSpec((16, 32), lambda i: (0, 0)),
            pl.BlockSpec((32, 32), lambda i: (0, 0)),
            pl.BlockSpec((1, 32), lambda i: (0, 0)),
            pl.BlockSpec((32, 16), lambda i: (0, 0)),
            pl.BlockSpec((1, 16), lambda i: (0, 0)),
        ],
        out_specs=[
            pl.BlockSpec((bn, 3), lambda i: (i, 0)),
            pl.BlockSpec((bn, 16), lambda i: (i, 0)),
            pl.BlockSpec((8, 128), lambda i: (0, 0)),
        ],
        out_shape=[
            jax.ShapeDtypeStruct((n, 3), jnp.float32),
            jax.ShapeDtypeStruct((n, 16), jnp.float32),
            jax.ShapeDtypeStruct((8, 128), jnp.float32),
        ],
    )(aggm, aggtc, coords, hidden, Wn1[:16], Wn1[16:], bn1.reshape(1, -1),
      Wn2, bn2.reshape(1, -1))


def _norm_body(hout_ref, stats_ref, out_ref, *, n):
    s = stats_ref[...]
    colsum = s[0:1, 0:16]
    sumsq = s[0:1, 16:17]
    mu = colsum / n
    var = sumsq / n - jnp.sum(mu * mu).reshape(1, 1)
    inv = jax.lax.rsqrt(1e-6 + var)
    out_ref[...] = (hout_ref[...] - mu) * inv


def _normalize(hout, stats):
    n = hout.shape[0]
    bn = _largest_block(n, 4000, 8)
    grid = n // bn
    return pl.pallas_call(
        functools.partial(_norm_body, n=float(n)),
        grid=(grid,),
        in_specs=[
            pl.BlockSpec((bn, 16), lambda i: (i, 0)),
            pl.BlockSpec((8, 128), lambda i: (0, 0)),
        ],
        out_specs=pl.BlockSpec((bn, 16), lambda i: (i, 0)),
        out_shape=jax.ShapeDtypeStruct((n, 16), jnp.float32),
    )(hout, stats)


def kernel(batch_coords, batch_hidden, edges, We1, be1, We2, be2, Wa, ba,
           Wc1, bc1, Wc2, Wn1, bn1, Wn2, bn2):
    n = batch_coords.shape[0]
    row = edges[0]
    table = jnp.pad(jnp.concatenate([batch_hidden, batch_coords], axis=1),
                    ((0, 0), (0, 109)))
    gr, gc = _sc_gather(table, edges[0:1], edges[1:2])
    payload = _edge_mlp(gr, gc, We1, be1, We2, be2, Wa, ba,
                        Wc1, bc1, Wc2)
    half = n // 2
    trash = jnp.int32(half)
    idx0 = jnp.where(row < half, row, trash).reshape(1, -1)
    idx1 = jnp.where(row >= half, row - half, trash).reshape(1, -1)
    zeros32 = jnp.zeros((8, 32), jnp.float32)
    zeros16 = jnp.zeros((8, 16), jnp.float32)
    aggpm = _sc_scatter(payload, idx0, idx1, zeros32, 32, 0)
    aggptc = _sc_scatter(payload, idx0, idx1, zeros16, 16, 2)
    aggm = jnp.concatenate([aggpm[:half, :32],
                            aggpm[_SC_HALF:_SC_HALF + half, :32]], axis=0)
    aggtc = jnp.concatenate([aggptc[:half, :16],
                             aggptc[_SC_HALF:_SC_HALF + half, :16]], axis=0)
    coords_out, h_out, stats = _node_update(
        aggm, aggtc, batch_coords, batch_hidden, Wn1, bn1, Wn2, bn2)
    h_norm = _normalize(h_out, stats)
    return coords_out, h_norm
